# Initial kernel scaffold; baseline (speedup 1.0000x reference)
import jax
import jax.numpy as jnp
from jax.experimental import pallas as pl


def kernel(node_feature, edge_index, edge_type, ef_init, Wmsg, bmsg, Wupd0, bupd0, Wupd, bupd):
    raise NotImplementedError("write your pallas kernel here")


# speedup vs baseline: 5.0446x; 5.0446x over previous
import functools

import jax
import jax.numpy as jnp
from jax import lax
from jax.experimental import pallas as pl
from jax.experimental.pallas import tpu as pltpu
from jax.experimental.pallas import tpu_sc as plsc

_NC = 2
_NS = 16
_CHUNK = 128


def _cdiv(a, b):
    return (a + b - 1) // b


def _tables_body(h_ref, ws_ref, wd_ref, b_ref, ps_ref, pd_ref):
    h = h_ref[...]
    for t in range(ws_ref.shape[0]):
        ps_ref[t] = jnp.dot(h, ws_ref[t], preferred_element_type=jnp.float32)
        pd_ref[t] = (jnp.dot(h, wd_ref[t], preferred_element_type=jnp.float32)
                     + b_ref[t][None, :])


@functools.lru_cache(maxsize=None)
def _tables_call(n, t, d, h, bn):
    return pl.pallas_call(
        _tables_body,
        grid=(n // bn,),
        in_specs=[
            pl.BlockSpec((bn, d), lambda i: (i, 0)),
            pl.BlockSpec((t, d, h), lambda i: (0, 0, 0)),
            pl.BlockSpec((t, d, h), lambda i: (0, 0, 0)),
            pl.BlockSpec((t, h), lambda i: (0, 0)),
        ],
        out_specs=[
            pl.BlockSpec((t, bn, h), lambda i: (0, i, 0)),
            pl.BlockSpec((t, bn, h), lambda i: (0, i, 0)),
        ],
        out_shape=[jax.ShapeDtypeStruct((t, n, h), jnp.float32)] * 2,
    )


def _et_body(ef_ref, tc_ref, we_ref, o_ref):
    ef = ef_ref[...]
    tt = tc_ref[...]
    acc = None
    for t in range(we_ref.shape[0]):
        m = (tt == float(t)).astype(jnp.float32)
        v = jnp.dot(ef, we_ref[t], preferred_element_type=jnp.float32) * m
        acc = v if acc is None else acc + v
    o_ref[...] = acc


@functools.lru_cache(maxsize=None)
def _et_call(ep, t, ef_dim, h, be):
    return pl.pallas_call(
        _et_body,
        grid=(ep // be,),
        in_specs=[
            pl.BlockSpec((be, ef_dim), lambda i: (i, 0)),
            pl.BlockSpec((be, 1), lambda i: (i, 0)),
            pl.BlockSpec((t, ef_dim, h), lambda i: (0, 0, 0)),
        ],
        out_specs=pl.BlockSpec((be, h), lambda i: (i, 0)),
        out_shape=jax.ShapeDtypeStruct((ep, h), jnp.float32),
    )


def _upd_body(a_ref, nf_ref, w1_ref, w2_ref, b_ref, o_ref, *, relu):
    o = (jnp.dot(a_ref[0] + a_ref[1], w1_ref[...],
                 preferred_element_type=jnp.float32)
         + jnp.dot(nf_ref[...], w2_ref[...],
                   preferred_element_type=jnp.float32)
         + b_ref[...])
    o_ref[...] = jnp.maximum(o, 0.0) if relu else o


@functools.lru_cache(maxsize=None)
def _upd_call(n, h, nf_dim, out_dim, bn, relu):
    return pl.pallas_call(
        functools.partial(_upd_body, relu=relu),
        grid=(n // bn,),
        in_specs=[
            pl.BlockSpec((2, bn, h), lambda i: (0, i, 0)),
            pl.BlockSpec((bn, nf_dim), lambda i: (i, 0)),
            pl.BlockSpec((h, out_dim), lambda i: (0, 0)),
            pl.BlockSpec((nf_dim, out_dim), lambda i: (0, 0)),
            pl.BlockSpec((1, out_dim), lambda i: (0, 0)),
        ],
        out_specs=pl.BlockSpec((bn, out_dim), lambda i: (i, 0)),
        out_shape=jax.ShapeDtypeStruct((n, out_dim), jnp.float32),
    )


@functools.lru_cache(maxsize=None)
def _make_sc_agg(n_pad, h, ne_tile):
    n_chunks = ne_tile // _CHUNK
    stripe = n_pad // _NS
    mesh = plsc.VectorSubcoreMesh(core_axis_name="c", subcore_axis_name="s")

    def body(psrc, pdst, et, gsrc, gdst, dsti, zrs, out,
             agg_sh, idx_s, idx_d, idx_o, buf_s, buf_d, buf_m, sem):
        cid = lax.axis_index("c")
        sid = lax.axis_index("s")
        wid = cid * _NS + sid
        pltpu.sync_copy(zrs, agg_sh.at[pl.ds(sid * stripe, stripe)])
        plsc.subcore_barrier()
        tile_base = wid * ne_tile

        def chunk_body(g, carry):
            base = tile_base + g * _CHUNK
            pltpu.sync_copy(gsrc.at[pl.ds(base, _CHUNK)], idx_s)
            pltpu.sync_copy(gdst.at[pl.ds(base, _CHUNK)], idx_d)
            pltpu.sync_copy(dsti.at[pl.ds(base, _CHUNK)], idx_o)
            cs = pltpu.async_copy(psrc.at[idx_s], buf_s, sem)
            cd = pltpu.async_copy(pdst.at[idx_d], buf_d, sem)
            cm = pltpu.async_copy(et.at[pl.ds(base, _CHUNK)], buf_m, sem)
            cs.wait()
            cd.wait()
            cm.wait()

            def row(r, c2):
                for cc in range(h // 16):
                    sl = pl.ds(cc * 16, 16)
                    buf_m[r, sl] = jnp.maximum(
                        buf_s[r, sl] + buf_d[r, sl] + buf_m[r, sl], 0.0)
                return c2

            lax.fori_loop(0, _CHUNK, row, 0, unroll=2)
            pltpu.sync_copy(buf_m, agg_sh.at[idx_o], add=True)
            return carry

        lax.fori_loop(0, n_chunks, chunk_body, 0)
        plsc.subcore_barrier()
        pltpu.sync_copy(agg_sh.at[pl.ds(sid * stripe, stripe)],
                        out.at[pl.ds(cid * n_pad + sid * stripe, stripe)])

    return pl.kernel(
        body,
        out_type=jax.ShapeDtypeStruct((_NC * n_pad, h), jnp.float32),
        mesh=mesh,
        scratch_types=[
            pltpu.VMEM_SHARED((n_pad, h), jnp.float32),
            pltpu.VMEM((_CHUNK,), jnp.int32),
            pltpu.VMEM((_CHUNK,), jnp.int32),
            pltpu.VMEM((_CHUNK,), jnp.int32),
            pltpu.VMEM((_CHUNK, h), jnp.float32),
            pltpu.VMEM((_CHUNK, h), jnp.float32),
            pltpu.VMEM((_CHUNK, h), jnp.float32),
            pltpu.SemaphoreType.DMA,
        ],
    )


def kernel(node_feature, edge_index, edge_type, ef_init, Wmsg, bmsg,
           Wupd0, bupd0, Wupd, bupd):
    n, nf_dim = node_feature.shape
    e = edge_index.shape[1]
    l_count, t_count, d_msg, h = Wmsg.shape
    ef_dim = ef_init.shape[1]
    out_dim = Wupd.shape[2]

    src = edge_index[0]
    dst = edge_index[1]
    et_i = edge_type.astype(jnp.int32)
    gsrc = et_i * n + src
    gdst = et_i * n + dst

    ep = _cdiv(e, _NC * _NS * _CHUNK) * (_NC * _NS * _CHUNK)
    pad = ep - e
    gsrc = jnp.concatenate([gsrc, jnp.zeros((pad,), jnp.int32)])
    gdst = jnp.concatenate([gdst, jnp.zeros((pad,), jnp.int32)])
    dstp = jnp.concatenate([dst, jnp.full((pad,), n, jnp.int32)])
    efp = jnp.concatenate([ef_init, jnp.zeros((pad, ef_dim), jnp.float32)])
    tcol = jnp.concatenate([edge_type.astype(jnp.float32),
                            jnp.zeros((pad,), jnp.float32)]).reshape(ep, 1)

    n_pad = 8 * _NS * _cdiv(n + 1, 8 * _NS)
    zrs = jnp.zeros((n_pad // _NS, h), jnp.float32)

    ws_all = Wmsg[:, :, :nf_dim, :]
    wd_all = Wmsg[:, :, nf_dim:2 * nf_dim, :]
    we_all = Wmsg[:, :, 2 * nf_dim:, :]
    w2_zero = jnp.zeros((nf_dim, out_dim), jnp.float32)

    bn = 2000
    be = 4096
    tables = _tables_call(n, t_count, nf_dim, h, bn)
    etc = _et_call(ep, t_count, ef_dim, h, be)
    sc_agg = _make_sc_agg(n_pad, h, ep // (_NC * _NS))

    cur = node_feature
    for l in range(l_count):
        ps, pd = tables(cur, ws_all[l], wd_all[l], bmsg[l])
        ps = ps.reshape(t_count * n, h)
        pd = pd.reshape(t_count * n, h)
        et_rows = etc(efp, tcol, we_all[l])
        aggp = sc_agg(ps, pd, et_rows, gsrc, gdst, dstp, zrs)
        aggp = aggp.reshape(_NC, n_pad, h)
        if l == 0:
            w1, w2, b = Wupd0, w2_zero, bupd0.reshape(1, out_dim)
        else:
            w1 = Wupd[l - 1][:h]
            w2 = Wupd[l - 1][h:]
            b = bupd[l - 1].reshape(1, out_dim)
        cur = _upd_call(n, h, nf_dim, out_dim, bn, l < l_count - 1)(
            aggp, node_feature, w1, w2, b)
    return cur


# speedup vs baseline: 8.0728x; 1.6003x over previous
import functools

import jax
import jax.numpy as jnp
from jax import lax
from jax.experimental import pallas as pl
from jax.experimental.pallas import tpu as pltpu
from jax.experimental.pallas import tpu_sc as plsc

_NC = 2
_NS = 16
_CHUNK = 56


def _cdiv(a, b):
    return (a + b - 1) // b


def _tables_body(h_ref, ws_ref, wd_ref, b_ref, ps_ref, pd_ref):
    h = h_ref[...]
    for t in range(ws_ref.shape[0]):
        ps_ref[t] = jnp.dot(h, ws_ref[t], preferred_element_type=jnp.float32)
        pd_ref[t] = (jnp.dot(h, wd_ref[t], preferred_element_type=jnp.float32)
                     + b_ref[t][None, :])


@functools.lru_cache(maxsize=None)
def _tables_call(n, t, d, h, bn):
    return pl.pallas_call(
        _tables_body,
        grid=(n // bn,),
        in_specs=[
            pl.BlockSpec((bn, d), lambda i: (i, 0)),
            pl.BlockSpec((t, d, h), lambda i: (0, 0, 0)),
            pl.BlockSpec((t, d, h), lambda i: (0, 0, 0)),
            pl.BlockSpec((t, h), lambda i: (0, 0)),
        ],
        out_specs=[
            pl.BlockSpec((t, bn, h), lambda i: (0, i, 0)),
            pl.BlockSpec((t, bn, h), lambda i: (0, i, 0)),
        ],
        out_shape=[jax.ShapeDtypeStruct((t, n, h), jnp.float32)] * 2,
    )


def _et_body(ef_ref, tc_ref, we_ref, o_ref):
    ef = ef_ref[...]
    tt = tc_ref[...]
    acc = None
    for t in range(we_ref.shape[0]):
        m = (tt == float(t)).astype(jnp.float32)
        v = jnp.dot(ef, we_ref[t], preferred_element_type=jnp.float32) * m
        acc = v if acc is None else acc + v
    o_ref[...] = acc


@functools.lru_cache(maxsize=None)
def _et_call(ep, t, ef_dim, h, be):
    return pl.pallas_call(
        _et_body,
        grid=(ep // be,),
        in_specs=[
            pl.BlockSpec((be, ef_dim), lambda i: (i, 0)),
            pl.BlockSpec((be, 1), lambda i: (i, 0)),
            pl.BlockSpec((t, ef_dim, h), lambda i: (0, 0, 0)),
        ],
        out_specs=pl.BlockSpec((be, h), lambda i: (i, 0)),
        out_shape=jax.ShapeDtypeStruct((ep, h), jnp.float32),
    )


def _upd_body(a_ref, nf_ref, w1_ref, w2_ref, b_ref, o_ref, *, relu):
    o = (jnp.dot(a_ref[0] + a_ref[1], w1_ref[...],
                 preferred_element_type=jnp.float32)
         + jnp.dot(nf_ref[...], w2_ref[...],
                   preferred_element_type=jnp.float32)
         + b_ref[...])
    o_ref[...] = jnp.maximum(o, 0.0) if relu else o


@functools.lru_cache(maxsize=None)
def _upd_call(n, h, nf_dim, out_dim, bn, relu):
    return pl.pallas_call(
        functools.partial(_upd_body, relu=relu),
        grid=(n // bn,),
        in_specs=[
            pl.BlockSpec((2, bn, h), lambda i: (0, i, 0)),
            pl.BlockSpec((bn, nf_dim), lambda i: (i, 0)),
            pl.BlockSpec((h, out_dim), lambda i: (0, 0)),
            pl.BlockSpec((nf_dim, out_dim), lambda i: (0, 0)),
            pl.BlockSpec((1, out_dim), lambda i: (0, 0)),
        ],
        out_specs=pl.BlockSpec((bn, out_dim), lambda i: (i, 0)),
        out_shape=jax.ShapeDtypeStruct((n, out_dim), jnp.float32),
    )


@functools.lru_cache(maxsize=None)
def _make_sc_agg(n_pad, h, ne_tile):
    n_chunks = ne_tile // _CHUNK
    assert n_chunks % 4 == 0 and n_chunks >= 12
    n_super = n_chunks // 4
    stripe = n_pad // 4
    mesh = plsc.VectorSubcoreMesh(core_axis_name="c", subcore_axis_name="s")

    def body(psrc, pdst, et, gidx, oidx, zrs, out,
             agg_sh, idxb, idxo, buf_s, buf_d, buf_m,
             sa0, sa1, sa2, sa3, sg0, sg1, ss0, ss1):
        sem_a = (sa0, sa1, sa2, sa3)
        sem_g = (sg0, sg1)
        sem_s = (ss0, ss1)
        cid = lax.axis_index("c")
        sid = lax.axis_index("s")
        wid = cid * _NS + sid

        @pl.when(sid < 4)
        def _():
            pltpu.sync_copy(zrs, agg_sh.at[pl.ds(sid * stripe, stripe)])

        plsc.subcore_barrier()
        row_base = wid * n_chunks

        def a_copies(g, j):
            return (
                pltpu.make_async_copy(gidx.at[row_base + g],
                                      idxb.at[j], sem_a[j]),
                pltpu.make_async_copy(oidx.at[row_base + g],
                                      idxo.at[j], sem_a[j]),
            )

        def b_copies(g, j, d):
            eb = (row_base + g) * _CHUNK
            return (
                pltpu.make_async_copy(psrc.at[idxb.at[j, 0]],
                                      buf_s.at[d], sem_g[d]),
                pltpu.make_async_copy(pdst.at[idxb.at[j, 1]],
                                      buf_d.at[d], sem_g[d]),
                pltpu.make_async_copy(et.at[pl.ds(eb, _CHUNK)],
                                      buf_m.at[d], sem_g[d]),
            )

        def d_copy(g, j, d):
            return pltpu.make_async_copy(buf_m.at[d],
                                         agg_sh.at[idxo.at[j]], sem_s[d])

        def compute(d):
            def row(r, carry):
                for cc in range(h // 16):
                    sl = pl.ds(cc * 16, 16)
                    buf_m[d, r, sl] = jnp.maximum(
                        buf_s[d, r, sl] + buf_d[d, r, sl] + buf_m[d, r, sl],
                        0.0)
                return carry

            lax.fori_loop(0, _CHUNK, row, 0, unroll=4)

        def stage(g, j, d, first, has1, has2):
            if not first:
                d_copy(g - 1, (j - 1) % 4, 1 - d).wait()
            if has1:
                for c in a_copies(g + 1, (j + 1) % 4):
                    c.wait()
                for c in b_copies(g + 1, (j + 1) % 4, 1 - d):
                    c.start()
            if has2:
                for c in a_copies(g + 2, (j + 2) % 4):
                    c.start()
            for c in b_copies(g, j, d):
                c.wait()
            compute(d)
            pltpu.async_copy(buf_m.at[d], agg_sh.at[idxo.at[j]],
                             sem_s[d], add=True)

        for c in a_copies(0, 0) + a_copies(1, 1):
            c.start()
        for c in a_copies(0, 0):
            c.wait()
        for c in b_copies(0, 0, 0):
            c.start()
        for g in range(4):
            stage(g, g % 4, g % 2, g == 0, True, g + 2 < n_chunks)

        def super_body(i, carry):
            g0 = i * 4
            for j in range(4):
                stage(g0 + j, j, j % 2, False, True, True)
            return carry

        lax.fori_loop(1, n_super - 1, super_body, 0)

        for g in range(n_chunks - 4, n_chunks):
            stage(g, g % 4, g % 2, False, g + 1 < n_chunks, g + 2 < n_chunks)
        d_copy(n_chunks - 1, (n_chunks - 1) % 4, (n_chunks - 1) % 2).wait()

        plsc.subcore_barrier()

        @pl.when(sid < 4)
        def _():
            pltpu.sync_copy(agg_sh.at[pl.ds(sid * stripe, stripe)],
                            out.at[pl.ds(cid * n_pad + sid * stripe, stripe)])

    return pl.kernel(
        body,
        out_type=jax.ShapeDtypeStruct((_NC * n_pad, h), jnp.float32),
        mesh=mesh,
        scratch_types=[
            pltpu.VMEM_SHARED((n_pad, h), jnp.float32),
            pltpu.VMEM((4, 2, _CHUNK), jnp.int32),
            pltpu.VMEM((4, _CHUNK), jnp.int32),
            pltpu.VMEM((2, _CHUNK, h), jnp.float32),
            pltpu.VMEM((2, _CHUNK, h), jnp.float32),
            pltpu.VMEM((2, _CHUNK, h), jnp.float32),
        ] + [pltpu.SemaphoreType.DMA] * 8,
    )


def kernel(node_feature, edge_index, edge_type, ef_init, Wmsg, bmsg,
           Wupd0, bupd0, Wupd, bupd):
    n, nf_dim = node_feature.shape
    e = edge_index.shape[1]
    l_count, t_count, d_msg, h = Wmsg.shape
    ef_dim = ef_init.shape[1]
    out_dim = Wupd.shape[2]

    src = edge_index[0]
    dst = edge_index[1]
    et_i = edge_type.astype(jnp.int32)
    gsrc = et_i * n + src
    gdst = et_i * n + dst

    grain = _NC * _NS * _CHUNK * 4
    ep = _cdiv(e, grain) * grain
    pad = ep - e
    gsrc = jnp.concatenate([gsrc, jnp.zeros((pad,), jnp.int32)])
    gdst = jnp.concatenate([gdst, jnp.zeros((pad,), jnp.int32)])
    dstp = jnp.concatenate([dst, jnp.full((pad,), n, jnp.int32)])
    gidx = jnp.stack([gsrc.reshape(-1, _CHUNK), gdst.reshape(-1, _CHUNK)],
                     axis=1)
    oidx = dstp.reshape(-1, _CHUNK)
    efp = jnp.concatenate([ef_init, jnp.zeros((pad, ef_dim), jnp.float32)])
    tcol = jnp.concatenate([edge_type.astype(jnp.float32),
                            jnp.zeros((pad,), jnp.float32)]).reshape(ep, 1)

    n_pad = 32 * _cdiv(n + 1, 32)
    zrs = jnp.zeros((n_pad // 4, h), jnp.float32)

    ws_all = Wmsg[:, :, :nf_dim, :]
    wd_all = Wmsg[:, :, nf_dim:2 * nf_dim, :]
    we_all = Wmsg[:, :, 2 * nf_dim:, :]
    w2_zero = jnp.zeros((nf_dim, out_dim), jnp.float32)

    bn = 2000
    be = _CHUNK * 64
    tables = _tables_call(n, t_count, nf_dim, h, bn)
    etc = _et_call(ep, t_count, ef_dim, h, be)
    sc_agg = _make_sc_agg(n_pad, h, ep // (_NC * _NS))

    cur = node_feature
    for l in range(l_count):
        ps, pd = tables(cur, ws_all[l], wd_all[l], bmsg[l])
        ps = ps.reshape(t_count * n, h)
        pd = pd.reshape(t_count * n, h)
        et_rows = etc(efp, tcol, we_all[l])
        aggp = sc_agg(ps, pd, et_rows, gidx, oidx, zrs)
        aggp = aggp.reshape(_NC, n_pad, h)
        if l == 0:
            w1, w2, b = Wupd0, w2_zero, bupd0.reshape(1, out_dim)
        else:
            w1 = Wupd[l - 1][:h]
            w2 = Wupd[l - 1][h:]
            b = bupd[l - 1].reshape(1, out_dim)
        cur = _upd_call(n, h, nf_dim, out_dim, bn, l < l_count - 1)(
            aggp, node_feature, w1, w2, b)
    return cur
